# baseline (speedup 1.0000x reference)
import functools

import jax
import jax.numpy as jnp
from jax import lax
from jax.experimental import pallas as pl
from jax.experimental.pallas import tpu as pltpu
from jax.experimental.pallas import tpu_sc as plsc

N = 10000
E = 320000
D_IN = 128
D_HID = 128
D_OUT = 64

NC = 2
NS = 16
NW = NC * NS
K = 128
NCH = 80
EPW = NCH * K
E_PAD = NW * EPW
NPAD = N + 8
RPT = 624
REM = N - RPT * NS

_f32 = jnp.float32


NPH = 2
CPP = NCH // NPH


def _zero_acc(s, zrow, acc):
  off = pl.multiple_of(s * RPT, 8)
  pltpu.sync_copy(zrow.at[pl.ds(off, RPT)], acc.at[pl.ds(off, RPT)])

  @pl.when(s == NS - 1)
  def _():
    pltpu.sync_copy(zrow.at[pl.ds(0, REM + 8)],
                    acc.at[pl.ds(RPT * NS, REM + 8)])

  return off


def _copy_out(c, s, off, acc, out):
  pltpu.sync_copy(acc.at[pl.ds(off, RPT)], out.at[c, pl.ds(off, RPT)])

  @pl.when(s == NS - 1)
  def _():
    pltpu.sync_copy(acc.at[pl.ds(RPT * NS, REM)],
                    out.at[c, pl.ds(RPT * NS, REM)])


def _make_sc_agg(D):
  mesh = plsc.VectorSubcoreMesh(core_axis_name="c", subcore_axis_name="s")
  out_type = [jax.ShapeDtypeStruct((NC, N, D), _f32)]
  KH = K // 2
  scratch = [
      pltpu.VMEM((CPP, K), jnp.int32),
      pltpu.VMEM((CPP, K), jnp.int32),
      pltpu.VMEM((2, K, D), _f32),
      pltpu.VMEM_SHARED((NPAD, D), _f32),
      pltpu.SemaphoreType.DMA,
      pltpu.SemaphoreType.DMA,
      pltpu.SemaphoreType.DMA,
      pltpu.SemaphoreType.DMA,
  ]

  def body(z, src3, dst3, zrow, agg_out, srcv, dstv, rowsv, acc,
           g0, g1, t0, t1):
    c = lax.axis_index("c")
    s = lax.axis_index("s")
    wid = c * NS + s
    gsem = (g0, g1)
    ssem = (t0, t1)
    off = _zero_acc(s, zrow, acc)
    plsc.subcore_barrier()

    def gather(i, b):
      pltpu.async_copy(z.at[srcv.at[i, pl.ds(0, KH)]],
                       rowsv.at[b, pl.ds(0, KH)], gsem[b])
      pltpu.async_copy(z.at[srcv.at[i, pl.ds(KH, KH)]],
                       rowsv.at[b, pl.ds(KH, KH)], gsem[b])

    def gwait(i, b):
      pltpu.make_async_copy(z.at[srcv.at[i, pl.ds(0, KH)]],
                            rowsv.at[b, pl.ds(0, KH)], gsem[b]).wait()
      pltpu.make_async_copy(z.at[srcv.at[i, pl.ds(KH, KH)]],
                            rowsv.at[b, pl.ds(KH, KH)], gsem[b]).wait()

    def swait(i, b):
      pltpu.make_async_copy(rowsv.at[b], acc.at[dstv.at[i]],
                            ssem[b]).wait()

    for p in range(NPH):
      pltpu.sync_copy(src3.at[wid, pl.ds(p * CPP, CPP)], srcv)
      pltpu.sync_copy(dst3.at[wid, pl.ds(p * CPP, CPP)], dstv)
      gather(0, 0)

      def step(i, carry):
        for b in range(2):
          @pl.when(i % 2 == b)
          def _():
            gwait(i, b)
            pltpu.async_copy(rowsv.at[b], acc.at[dstv.at[i]], ssem[b],
                             add=True)

            @pl.when(i + 1 < CPP)
            def _():
              @pl.when(i >= 1)
              def _():
                swait(i - 1, 1 - b)

              gather(i + 1, 1 - b)

        return carry

      lax.fori_loop(0, CPP, step, 0)
      swait(CPP - 2, 0 if (CPP - 2) % 2 == 0 else 1)
      swait(CPP - 1, 0 if (CPP - 1) % 2 == 0 else 1)

    plsc.subcore_barrier()
    _copy_out(c, s, off, acc, agg_out)

  return pl.kernel(body, mesh=mesh, out_type=out_type, scratch_types=scratch)


def _make_sc_cnt():
  mesh = plsc.VectorSubcoreMesh(core_axis_name="c", subcore_axis_name="s")
  out_type = [jax.ShapeDtypeStruct((NC, N, D_HID), _f32)]
  scratch = [
      pltpu.VMEM((NCH, K), jnp.int32),
      pltpu.VMEM((K, D_HID), _f32),
      pltpu.VMEM_SHARED((NPAD, D_HID), _f32),
  ]

  def body(dst3, zrow, one_tab, cnt_out, dstv, onesv, acc):
    c = lax.axis_index("c")
    s = lax.axis_index("s")
    wid = c * NS + s
    pltpu.sync_copy(dst3.at[wid], dstv)
    pltpu.sync_copy(one_tab, onesv)
    off = _zero_acc(s, zrow, acc)
    plsc.subcore_barrier()

    def chunk(i, carry):
      pltpu.sync_copy(onesv, acc.at[dstv.at[i]], add=True)
      return carry

    lax.fori_loop(0, NCH, chunk, 0)
    plsc.subcore_barrier()
    _copy_out(c, s, off, acc, cnt_out)

  return pl.kernel(body, mesh=mesh, out_type=out_type, scratch_types=scratch)


_sc_agg_128 = functools.lru_cache(maxsize=None)(
    lambda: _make_sc_agg(D_HID))
_sc_cnt = functools.lru_cache(maxsize=None)(_make_sc_cnt)


BN = 1000


def _row_spec(d):
  return pl.BlockSpec((BN, d), lambda i: (i, 0))


def _full_spec(shape):
  return pl.BlockSpec(shape, lambda i: (0, 0))


def _dot(a, b):
  return jnp.dot(a, b, preferred_element_type=_f32,
                 precision=lax.Precision.HIGHEST)


def _tc_pre(x, wl, wr, bl):
  d_in, d_out = wl.shape

  def body(x_r, wl_r, wr_r, bl_r, z_r, r_r):
    xv = x_r[...]
    z_r[...] = _dot(xv, wl_r[...])
    r_r[...] = _dot(xv, wr_r[...]) + bl_r[...]

  return pl.pallas_call(
      body,
      grid=(N // BN,),
      in_specs=[_row_spec(d_in), _full_spec(wl.shape), _full_spec(wr.shape),
                _full_spec((1, d_out))],
      out_specs=[_row_spec(d_out), _row_spec(d_out)],
      out_shape=[jax.ShapeDtypeStruct((N, d_out), _f32)] * 2,
  )(x, wl, wr, bl.reshape(1, -1))


def _ln(pre, g, be):
  mu = jnp.mean(pre, axis=-1, keepdims=True)
  dv = pre - mu
  var = jnp.mean(dv * dv, axis=-1, keepdims=True)
  return dv / jnp.sqrt(var + 1e-5) * g + be


def _tc_mid(agg, cnta, cntb, r, g, be, wl, wr, bl, emit_h):
  d = r.shape[1]
  dz = d if emit_h else wl.shape[1]
  dr = wr.shape[1]

  def body(aa_r, ab_r, ca_r, cb_r, r_r, g_r, be_r, wl_r, wr_r, bl_r,
           z_r, rn_r):
    cnt = jnp.maximum(ca_r[...] + cb_r[...], 1.0)
    pre = (aa_r[...] + ab_r[...]) / cnt + r_r[...]
    h = jnp.maximum(_ln(pre, g_r[...], be_r[...]), 0.0)
    if emit_h:
      z_r[...] = h
    else:
      z_r[...] = _dot(h, wl_r[...])
    rn_r[...] = _dot(h, wr_r[...]) + bl_r[...]

  return pl.pallas_call(
      body,
      grid=(N // BN,),
      in_specs=[_row_spec(d), _row_spec(d),
                pl.BlockSpec((BN, 1), lambda i: (i, 0)),
                pl.BlockSpec((BN, 1), lambda i: (i, 0)),
                _row_spec(d), _full_spec((1, d)), _full_spec((1, d)),
                _full_spec(wl.shape), _full_spec(wr.shape),
                _full_spec((1, dr))],
      out_specs=[_row_spec(dz), _row_spec(dr)],
      out_shape=[jax.ShapeDtypeStruct((N, dz), _f32),
                 jax.ShapeDtypeStruct((N, dr), _f32)],
  )(agg[0], agg[1], cnta, cntb, r, g.reshape(1, -1), be.reshape(1, -1),
    wl, wr, bl.reshape(1, -1))


def _tc_post(agg, cnta, cntb, r, wl, g, be):
  d = agg.shape[2]
  do = r.shape[1]

  def body(aa_r, ab_r, ca_r, cb_r, r_r, wl_r, g_r, be_r, o_r):
    cnt = jnp.maximum(ca_r[...] + cb_r[...], 1.0)
    mean = (aa_r[...] + ab_r[...]) / cnt
    pre = _dot(mean, wl_r[...]) + r_r[...]
    h = _ln(pre, g_r[...], be_r[...])
    nrm = jnp.sqrt(jnp.sum(h * h, axis=-1, keepdims=True))
    o_r[...] = h / jnp.maximum(nrm, 1e-12)

  return pl.pallas_call(
      body,
      grid=(N // BN,),
      in_specs=[_row_spec(d), _row_spec(d),
                pl.BlockSpec((BN, 1), lambda i: (i, 0)),
                pl.BlockSpec((BN, 1), lambda i: (i, 0)),
                _row_spec(do), _full_spec(wl.shape),
                _full_spec((1, do)), _full_spec((1, do))],
      out_specs=_row_spec(do),
      out_shape=jax.ShapeDtypeStruct((N, do), _f32),
  )(agg[0], agg[1], cnta, cntb, r, wl, g.reshape(1, -1), be.reshape(1, -1))


def kernel(x, edge_index, Wl0, bl0, Wr0, g0, be0, Wl1, bl1, Wr1, g1, be1,
           Wl2, bl2, Wr2, g2, be2):
  npad = E_PAD - E
  src3 = jnp.concatenate(
      [edge_index[0], jnp.zeros((npad,), jnp.int32)]).reshape(NW, NCH, K)
  dst3 = jnp.concatenate(
      [edge_index[1], jnp.full((npad,), N, jnp.int32)]).reshape(NW, NCH, K)
  zrow = jnp.zeros((N, D_HID), _f32)
  ones_blk = jnp.ones((K, D_HID), _f32)
  sc_agg = _sc_agg_128()

  (cntw,) = _sc_cnt()(dst3, zrow, ones_blk)
  cnta = cntw[0, :, 0:1]
  cntb = cntw[1, :, 0:1]

  z0, r0 = _tc_pre(x, Wl0, Wr0, bl0)
  (agg0,) = sc_agg(z0, src3, dst3, zrow)
  z1, r1 = _tc_mid(agg0, cnta, cntb, r0, g0, be0, Wl1, Wr1, bl1,
                   emit_h=False)

  (agg1,) = sc_agg(z1, src3, dst3, zrow)
  h2, r2 = _tc_mid(agg1, cnta, cntb, r1, g1, be1, Wl2, Wr2, bl2,
                   emit_h=True)

  (agg2,) = sc_agg(h2, src3, dst3, zrow)
  return _tc_post(agg2, cnta, cntb, r2, Wl2, g2, be2)


# speedup vs baseline: 1.1411x; 1.1411x over previous
import functools

import jax
import jax.numpy as jnp
from jax import lax
from jax.experimental import pallas as pl
from jax.experimental.pallas import tpu as pltpu
from jax.experimental.pallas import tpu_sc as plsc

N = 10000
E = 320000
D_IN = 128
D_HID = 128
D_OUT = 64
DH = D_HID // 2

NC = 2
NS = 16
NW = NC * NS
K = 128
NCH = 80
E_PAD = NW * NCH * K
TCH = E_PAD // (NS * K)
CPP = 40
NPAD = N + 8
RPT = 624
REM = N - RPT * NS

_f32 = jnp.float32


def _zero_acc(s, zrow, acc):
  off = pl.multiple_of(s * RPT, 8)
  pltpu.sync_copy(zrow.at[pl.ds(off, RPT)], acc.at[pl.ds(off, RPT)])

  @pl.when(s == NS - 1)
  def _():
    pltpu.sync_copy(zrow.at[pl.ds(0, REM + 8)],
                    acc.at[pl.ds(RPT * NS, REM + 8)])

  return off


def _stage_table(s, zh, table):
  off = pl.multiple_of(s * RPT, 8)
  pltpu.sync_copy(zh.at[pl.ds(off, RPT)], table.at[pl.ds(off, RPT)])

  @pl.when(s == NS - 1)
  def _():
    pltpu.sync_copy(zh.at[pl.ds(RPT * NS, REM)],
                    table.at[pl.ds(RPT * NS, REM)])


def _copy_out(c, s, off, acc, out):
  pltpu.sync_copy(acc.at[pl.ds(off, RPT)], out.at[c, pl.ds(off, RPT)])

  @pl.when(s == NS - 1)
  def _():
    pltpu.sync_copy(acc.at[pl.ds(RPT * NS, REM)],
                    out.at[c, pl.ds(RPT * NS, REM)])


def _agg_loop(table, acc, src_idx, dst_idx, srcv, dstv, rowsv, gsem,
              n_chunks, idx_row):
  for p in range(n_chunks // CPP):
    pltpu.sync_copy(src_idx.at[idx_row, pl.ds(p * CPP, CPP)], srcv)
    pltpu.sync_copy(dst_idx.at[idx_row, pl.ds(p * CPP, CPP)], dstv)
    pltpu.async_copy(table.at[srcv.at[0]], rowsv.at[0], gsem[0])
    pltpu.async_copy(table.at[srcv.at[1]], rowsv.at[1], gsem[1])

    def pair(j, carry):
      for b in range(2):
        i = j * 2 + b
        pltpu.make_async_copy(table.at[srcv.at[i]], rowsv.at[b],
                              gsem[b]).wait()
        pltpu.sync_copy(rowsv.at[b], acc.at[dstv.at[i]], add=True)

        @pl.when(i + 2 < CPP)
        def _():
          pltpu.async_copy(table.at[srcv.at[i + 2]], rowsv.at[b], gsem[b])

      return carry

    lax.fori_loop(0, CPP // 2, pair, 0)


def _make_sc_agg_hbm128():
  mesh = plsc.VectorSubcoreMesh(core_axis_name="c", subcore_axis_name="s")
  out_type = [jax.ShapeDtypeStruct((NC, N, D_HID), _f32)]
  scratch = [
      pltpu.VMEM((CPP, K), jnp.int32),
      pltpu.VMEM((CPP, K), jnp.int32),
      pltpu.VMEM((2, K, D_HID), _f32),
      pltpu.VMEM_SHARED((NPAD, D_HID), _f32),
      pltpu.SemaphoreType.DMA,
      pltpu.SemaphoreType.DMA,
  ]

  def body(z, src3, dst3, zrow, agg_out, srcv, dstv, rowsv, acc, g0, g1):
    c = lax.axis_index("c")
    s = lax.axis_index("s")
    wid = c * NS + s
    off = _zero_acc(s, zrow, acc)
    plsc.subcore_barrier()
    _agg_loop(z, acc, src3, dst3, srcv, dstv, rowsv, (g0, g1), NCH, wid)
    plsc.subcore_barrier()
    _copy_out(c, s, off, acc, agg_out)

  return pl.kernel(body, mesh=mesh, out_type=out_type, scratch_types=scratch)


def _make_sc_cnt():
  mesh = plsc.VectorSubcoreMesh(core_axis_name="c", subcore_axis_name="s")
  out_type = [jax.ShapeDtypeStruct((NC, N, D_HID), _f32)]
  scratch = [
      pltpu.VMEM((NCH, K), jnp.int32),
      pltpu.VMEM((K, D_HID), _f32),
      pltpu.VMEM_SHARED((NPAD, D_HID), _f32),
  ]

  def body(dst3, zrow, one_tab, cnt_out, dstv, onesv, acc):
    c = lax.axis_index("c")
    s = lax.axis_index("s")
    wid = c * NS + s
    pltpu.sync_copy(dst3.at[wid], dstv)
    pltpu.sync_copy(one_tab, onesv)
    off = _zero_acc(s, zrow, acc)
    plsc.subcore_barrier()

    def chunk(i, carry):
      pltpu.sync_copy(onesv, acc.at[dstv.at[i]], add=True)
      return carry

    lax.fori_loop(0, NCH, chunk, 0)
    plsc.subcore_barrier()
    _copy_out(c, s, off, acc, cnt_out)

  return pl.kernel(body, mesh=mesh, out_type=out_type, scratch_types=scratch)


_sc_agg_hbm128 = functools.lru_cache(maxsize=None)(_make_sc_agg_hbm128)
_sc_cnt = functools.lru_cache(maxsize=None)(_make_sc_cnt)


BN = 1000


def _row_spec(d):
  return pl.BlockSpec((BN, d), lambda i: (i, 0))


def _full_spec(shape):
  return pl.BlockSpec(shape, lambda i: (0, 0))


def _dot(a, b):
  return jnp.dot(a, b, preferred_element_type=_f32,
                 precision=lax.Precision.HIGHEST)


def _ln(pre, g, be):
  mu = jnp.mean(pre, axis=-1, keepdims=True)
  dv = pre - mu
  var = jnp.mean(dv * dv, axis=-1, keepdims=True)
  return dv / jnp.sqrt(var + 1e-5) * g + be


def _tc_pre(x, wl, wr, bl):
  d_in, d_out = wl.shape

  def body(x_r, wl_r, wr_r, bl_r, z_r, r_r):
    xv = x_r[...]
    z_r[...] = _dot(xv, wl_r[...])
    r_r[...] = _dot(xv, wr_r[...]) + bl_r[...]

  return pl.pallas_call(
      body,
      grid=(N // BN,),
      in_specs=[_row_spec(d_in), _full_spec(wl.shape), _full_spec(wr.shape),
                _full_spec((1, d_out))],
      out_specs=[_row_spec(d_out), _row_spec(d_out)],
      out_shape=[jax.ShapeDtypeStruct((N, d_out), _f32)] * 2,
  )(x, wl, wr, bl.reshape(1, -1))


def _tc_mid(aggA, aggB, cnta, cntb, r, g, be, wl, wr, bl, combine,
            emit_h=False):
  d = r.shape[1]
  da = aggA.shape[1]
  dz = d if emit_h else wl.shape[1]
  dr = wr.shape[1]

  def body(aa_r, ab_r, ca_r, cb_r, r_r, g_r, be_r, wl_r, wr_r, bl_r,
           z_r, rn_r):
    cnt = jnp.maximum(ca_r[...] + cb_r[...], 1.0)
    if combine == "sum":
      agg = aa_r[...] + ab_r[...]
    else:
      agg = jnp.concatenate([aa_r[...], ab_r[...]], axis=-1)
    pre = agg / cnt + r_r[...]
    h = jnp.maximum(_ln(pre, g_r[...], be_r[...]), 0.0)
    if emit_h:
      z_r[...] = h
    else:
      z_r[...] = _dot(h, wl_r[...])
    rn_r[...] = _dot(h, wr_r[...]) + bl_r[...]

  return pl.pallas_call(
      body,
      grid=(N // BN,),
      in_specs=[_row_spec(da), _row_spec(da),
                pl.BlockSpec((BN, 1), lambda i: (i, 0)),
                pl.BlockSpec((BN, 1), lambda i: (i, 0)),
                _row_spec(d), _full_spec((1, d)), _full_spec((1, d)),
                _full_spec(wl.shape), _full_spec(wr.shape),
                _full_spec((1, dr))],
      out_specs=[_row_spec(dz), _row_spec(dr)],
      out_shape=[jax.ShapeDtypeStruct((N, dz), _f32),
                 jax.ShapeDtypeStruct((N, dr), _f32)],
  )(aggA, aggB, cnta, cntb, r, g.reshape(1, -1), be.reshape(1, -1),
    wl, wr, bl.reshape(1, -1))


def _tc_post(aggA, aggB, cnta, cntb, r, wl, g, be):
  da = aggA.shape[1]
  do = r.shape[1]

  def body(aa_r, ab_r, ca_r, cb_r, r_r, wl_r, g_r, be_r, o_r):
    cnt = jnp.maximum(ca_r[...] + cb_r[...], 1.0)
    mean = (aa_r[...] + ab_r[...]) / cnt
    pre = _dot(mean, wl_r[...]) + r_r[...]
    h = _ln(pre, g_r[...], be_r[...])
    nrm = jnp.sqrt(jnp.sum(h * h, axis=-1, keepdims=True))
    o_r[...] = h / jnp.maximum(nrm, 1e-12)

  return pl.pallas_call(
      body,
      grid=(N // BN,),
      in_specs=[_row_spec(da), _row_spec(da),
                pl.BlockSpec((BN, 1), lambda i: (i, 0)),
                pl.BlockSpec((BN, 1), lambda i: (i, 0)),
                _row_spec(do), _full_spec(wl.shape),
                _full_spec((1, do)), _full_spec((1, do))],
      out_specs=_row_spec(do),
      out_shape=jax.ShapeDtypeStruct((N, do), _f32),
  )(aggA, aggB, cnta, cntb, r, wl, g.reshape(1, -1), be.reshape(1, -1))


def kernel(x, edge_index, Wl0, bl0, Wr0, g0, be0, Wl1, bl1, Wr1, g1, be1,
           Wl2, bl2, Wr2, g2, be2):
  npad = E_PAD - E
  srcp = jnp.concatenate([edge_index[0], jnp.zeros((npad,), jnp.int32)])
  dstp = jnp.concatenate([edge_index[1], jnp.full((npad,), N, jnp.int32)])
  src3 = srcp.reshape(NW, NCH, K)
  dst3 = dstp.reshape(NW, NCH, K)
  ones_blk = jnp.ones((K, D_HID), _f32)

  (cntw,) = _sc_cnt()(dst3, jnp.zeros((N, D_HID), _f32), ones_blk)
  cnta = cntw[0, :, 0:1]
  cntb = cntw[1, :, 0:1]
  zrow128 = cntw[0] * 0.0

  z0, r0 = _tc_pre(x, Wl0, Wr0, bl0)
  (agg0,) = _sc_agg_hbm128()(z0, src3, dst3, zrow128)
  z1, r1 = _tc_mid(agg0[0], agg0[1], cnta, cntb, r0, g0, be0,
                   Wl1, Wr1, bl1, combine="sum")

  (agg1,) = _sc_agg_hbm128()(z1, src3, dst3, zrow128)
  h2, r2 = _tc_mid(agg1[0], agg1[1], cnta, cntb, r1, g1, be1,
                   Wl2, Wr2, bl2, combine="sum", emit_h=True)

  (agg2,) = _sc_agg_hbm128()(h2, src3, dst3, zrow128)
  return _tc_post(agg2[0], agg2[1], cnta, cntb, r2, Wl2, g2, be2)


# speedup vs baseline: 1.2284x; 1.0765x over previous
import functools

import jax
import jax.numpy as jnp
from jax import lax
from jax.experimental import pallas as pl
from jax.experimental.pallas import tpu as pltpu
from jax.experimental.pallas import tpu_sc as plsc

N = 10000
E = 320000
D_IN = 128
D_HID = 128
D_OUT = 64
DH = D_HID // 2

NC = 2
NS = 16
NW = NC * NS
K = 128
NCH = 80
E_PAD = NW * NCH * K
TCH = E_PAD // (NS * K)
CPP = 40
NPAD = N + 8
RPT = 624
REM = N - RPT * NS

_f32 = jnp.float32


def _zero_acc(s, zrow, acc):
  off = pl.multiple_of(s * RPT, 8)
  pltpu.sync_copy(zrow.at[pl.ds(off, RPT)], acc.at[pl.ds(off, RPT)])

  @pl.when(s == NS - 1)
  def _():
    pltpu.sync_copy(zrow.at[pl.ds(0, REM + 8)],
                    acc.at[pl.ds(RPT * NS, REM + 8)])

  return off


def _stage_table(s, zh, table):
  off = pl.multiple_of(s * RPT, 8)
  pltpu.sync_copy(zh.at[pl.ds(off, RPT)], table.at[pl.ds(off, RPT)])

  @pl.when(s == NS - 1)
  def _():
    pltpu.sync_copy(zh.at[pl.ds(RPT * NS, REM)],
                    table.at[pl.ds(RPT * NS, REM)])


def _copy_out(c, s, off, acc, out):
  pltpu.sync_copy(acc.at[pl.ds(off, RPT)], out.at[c, pl.ds(off, RPT)])

  @pl.when(s == NS - 1)
  def _():
    pltpu.sync_copy(acc.at[pl.ds(RPT * NS, REM)],
                    out.at[c, pl.ds(RPT * NS, REM)])


def _agg_loop(table, acc, src_idx, dst_idx, srcv, dstv, rowsv, gsem,
              n_chunks, idx_row):
  for p in range(n_chunks // CPP):
    pltpu.sync_copy(src_idx.at[idx_row, pl.ds(p * CPP, CPP)], srcv)
    pltpu.sync_copy(dst_idx.at[idx_row, pl.ds(p * CPP, CPP)], dstv)
    pltpu.async_copy(table.at[srcv.at[0]], rowsv.at[0], gsem[0])
    pltpu.async_copy(table.at[srcv.at[1]], rowsv.at[1], gsem[1])

    def pair(j, carry):
      for b in range(2):
        i = j * 2 + b
        pltpu.make_async_copy(table.at[srcv.at[i]], rowsv.at[b],
                              gsem[b]).wait()
        pltpu.sync_copy(rowsv.at[b], acc.at[dstv.at[i]], add=True)

        @pl.when(i + 2 < CPP)
        def _():
          pltpu.async_copy(table.at[srcv.at[i + 2]], rowsv.at[b], gsem[b])

      return carry

    lax.fori_loop(0, CPP // 2, pair, 0)


def _make_sc_agg_hbm128():
  mesh = plsc.VectorSubcoreMesh(core_axis_name="c", subcore_axis_name="s")
  out_type = [jax.ShapeDtypeStruct((NC, N, D_HID), _f32)]
  scratch = [
      pltpu.VMEM((CPP, K), jnp.int32),
      pltpu.VMEM((CPP, K), jnp.int32),
      pltpu.VMEM((2, K, D_HID), _f32),
      pltpu.VMEM_SHARED((NPAD, D_HID), _f32),
      pltpu.SemaphoreType.DMA,
      pltpu.SemaphoreType.DMA,
  ]

  def body(z, src3, dst3, zrow, agg_out, srcv, dstv, rowsv, acc, g0, g1):
    c = lax.axis_index("c")
    s = lax.axis_index("s")
    wid = c * NS + s
    off = _zero_acc(s, zrow, acc)
    plsc.subcore_barrier()
    _agg_loop(z, acc, src3, dst3, srcv, dstv, rowsv, (g0, g1), NCH, wid)
    plsc.subcore_barrier()
    _copy_out(c, s, off, acc, agg_out)

  return pl.kernel(body, mesh=mesh, out_type=out_type, scratch_types=scratch)


def _make_sc_agg_hbm64():
  mesh = plsc.VectorSubcoreMesh(core_axis_name="c", subcore_axis_name="s")
  out_type = [jax.ShapeDtypeStruct((NC, N, DH), _f32)]
  scratch = [
      pltpu.VMEM((CPP, K), jnp.int32),
      pltpu.VMEM((CPP, K), jnp.int32),
      pltpu.VMEM((2, K, DH), _f32),
      pltpu.VMEM_SHARED((NPAD, DH), _f32),
      pltpu.SemaphoreType.DMA,
      pltpu.SemaphoreType.DMA,
  ]

  def body(z, src3, dst3, zrow, agg_out, srcv, dstv, rowsv, acc, g0, g1):
    c = lax.axis_index("c")
    s = lax.axis_index("s")
    wid = c * NS + s
    off = _zero_acc(s, zrow, acc)
    plsc.subcore_barrier()
    _agg_loop(z, acc, src3, dst3, srcv, dstv, rowsv, (g0, g1), NCH, wid)
    plsc.subcore_barrier()
    _copy_out(c, s, off, acc, agg_out)

  return pl.kernel(
      body, mesh=mesh, out_type=out_type, scratch_types=scratch,
      compiler_params=pltpu.CompilerParams(use_tc_tiling_on_sc=False))


def _make_sc_cnt():
  mesh = plsc.VectorSubcoreMesh(core_axis_name="c", subcore_axis_name="s")
  out_type = [jax.ShapeDtypeStruct((NC, N, D_HID), _f32)]
  scratch = [
      pltpu.VMEM((NCH, K), jnp.int32),
      pltpu.VMEM((K, D_HID), _f32),
      pltpu.VMEM_SHARED((NPAD, D_HID), _f32),
  ]

  def body(dst3, zrow, one_tab, cnt_out, dstv, onesv, acc):
    c = lax.axis_index("c")
    s = lax.axis_index("s")
    wid = c * NS + s
    pltpu.sync_copy(dst3.at[wid], dstv)
    pltpu.sync_copy(one_tab, onesv)
    off = _zero_acc(s, zrow, acc)
    plsc.subcore_barrier()

    def chunk(i, carry):
      pltpu.sync_copy(onesv, acc.at[dstv.at[i]], add=True)
      return carry

    lax.fori_loop(0, NCH, chunk, 0)
    plsc.subcore_barrier()
    _copy_out(c, s, off, acc, cnt_out)

  return pl.kernel(body, mesh=mesh, out_type=out_type, scratch_types=scratch)


_sc_agg_hbm128 = functools.lru_cache(maxsize=None)(_make_sc_agg_hbm128)
_sc_agg_hbm64 = functools.lru_cache(maxsize=None)(_make_sc_agg_hbm64)
_sc_cnt = functools.lru_cache(maxsize=None)(_make_sc_cnt)


BN = 1000


def _row_spec(d):
  return pl.BlockSpec((BN, d), lambda i: (i, 0))


def _full_spec(shape):
  return pl.BlockSpec(shape, lambda i: (0, 0))


def _dot(a, b):
  return jnp.dot(a, b, preferred_element_type=_f32,
                 precision=lax.Precision.HIGHEST)


def _ln(pre, g, be):
  mu = jnp.mean(pre, axis=-1, keepdims=True)
  dv = pre - mu
  var = jnp.mean(dv * dv, axis=-1, keepdims=True)
  return dv / jnp.sqrt(var + 1e-5) * g + be


def _tc_pre(x, wl, wr, bl):
  d_in, d_out = wl.shape

  def body(x_r, wl_r, wr_r, bl_r, z_r, r_r):
    xv = x_r[...]
    z_r[...] = _dot(xv, wl_r[...])
    r_r[...] = _dot(xv, wr_r[...]) + bl_r[...]

  return pl.pallas_call(
      body,
      grid=(N // BN,),
      in_specs=[_row_spec(d_in), _full_spec(wl.shape), _full_spec(wr.shape),
                _full_spec((1, d_out))],
      out_specs=[_row_spec(d_out), _row_spec(d_out)],
      out_shape=[jax.ShapeDtypeStruct((N, d_out), _f32)] * 2,
  )(x, wl, wr, bl.reshape(1, -1))


def _tc_mid(aggA, aggB, cnta, cntb, r, g, be, wl, wr, bl, combine,
            emit_h=False):
  d = r.shape[1]
  da = aggA.shape[1]
  dz = d if emit_h else wl.shape[1]
  dr = wr.shape[1]

  def body(aa_r, ab_r, ca_r, cb_r, r_r, g_r, be_r, wl_r, wr_r, bl_r,
           z_r, rn_r):
    cnt = jnp.maximum(ca_r[...] + cb_r[...], 1.0)
    if combine == "sum":
      agg = aa_r[...] + ab_r[...]
    else:
      agg = jnp.concatenate([aa_r[...], ab_r[...]], axis=-1)
    pre = agg / cnt + r_r[...]
    h = jnp.maximum(_ln(pre, g_r[...], be_r[...]), 0.0)
    if emit_h:
      z_r[...] = h
    else:
      z_r[...] = _dot(h, wl_r[...])
    rn_r[...] = _dot(h, wr_r[...]) + bl_r[...]

  return pl.pallas_call(
      body,
      grid=(N // BN,),
      in_specs=[_row_spec(da), _row_spec(da),
                pl.BlockSpec((BN, 1), lambda i: (i, 0)),
                pl.BlockSpec((BN, 1), lambda i: (i, 0)),
                _row_spec(d), _full_spec((1, d)), _full_spec((1, d)),
                _full_spec(wl.shape), _full_spec(wr.shape),
                _full_spec((1, dr))],
      out_specs=[_row_spec(dz), _row_spec(dr)],
      out_shape=[jax.ShapeDtypeStruct((N, dz), _f32),
                 jax.ShapeDtypeStruct((N, dr), _f32)],
  )(aggA, aggB, cnta, cntb, r, g.reshape(1, -1), be.reshape(1, -1),
    wl, wr, bl.reshape(1, -1))


def _tc_post(aggA, aggB, cnta, cntb, r, g, be):
  do = r.shape[1]

  def body(aa_r, ab_r, ca_r, cb_r, r_r, g_r, be_r, o_r):
    cnt = jnp.maximum(ca_r[...] + cb_r[...], 1.0)
    pre = (aa_r[...] + ab_r[...]) / cnt + r_r[...]
    h = _ln(pre, g_r[...], be_r[...])
    nrm = jnp.sqrt(jnp.sum(h * h, axis=-1, keepdims=True))
    o_r[...] = h / jnp.maximum(nrm, 1e-12)

  return pl.pallas_call(
      body,
      grid=(N // BN,),
      in_specs=[_row_spec(do), _row_spec(do),
                pl.BlockSpec((BN, 1), lambda i: (i, 0)),
                pl.BlockSpec((BN, 1), lambda i: (i, 0)),
                _row_spec(do), _full_spec((1, do)), _full_spec((1, do))],
      out_specs=_row_spec(do),
      out_shape=jax.ShapeDtypeStruct((N, do), _f32),
  )(aggA, aggB, cnta, cntb, r, g.reshape(1, -1), be.reshape(1, -1))


def kernel(x, edge_index, Wl0, bl0, Wr0, g0, be0, Wl1, bl1, Wr1, g1, be1,
           Wl2, bl2, Wr2, g2, be2):
  npad = E_PAD - E
  srcp = jnp.concatenate([edge_index[0], jnp.zeros((npad,), jnp.int32)])
  dstp = jnp.concatenate([edge_index[1], jnp.full((npad,), N, jnp.int32)])
  src3 = srcp.reshape(NW, NCH, K)
  dst3 = dstp.reshape(NW, NCH, K)
  ones_blk = jnp.ones((K, D_HID), _f32)

  (cntw,) = _sc_cnt()(dst3, jnp.zeros((N, D_HID), _f32), ones_blk)
  cnta = cntw[0, :, 0:1]
  cntb = cntw[1, :, 0:1]
  zrow128 = cntw[0] * 0.0

  z0, r0 = _tc_pre(x, Wl0, Wr0, bl0)
  (agg0,) = _sc_agg_hbm128()(z0, src3, dst3, zrow128)
  z1, r1 = _tc_mid(agg0[0], agg0[1], cnta, cntb, r0, g0, be0,
                   Wl1, Wr1, bl1, combine="sum")

  (agg1,) = _sc_agg_hbm128()(z1, src3, dst3, zrow128)
  z2, r2 = _tc_mid(agg1[0], agg1[1], cnta, cntb, r1, g1, be1,
                   Wl2, Wr2, bl2, combine="sum")

  (agg2,) = _sc_agg_hbm64()(z2, src3, dst3, zrow128[:, :DH])
  return _tc_post(agg2[0], agg2[1], cnta, cntb, r2, g2, be2)


# speedup vs baseline: 1.7296x; 1.4079x over previous
import functools

import jax
import jax.numpy as jnp
from jax import lax
from jax.experimental import pallas as pl
from jax.experimental.pallas import tpu as pltpu
from jax.experimental.pallas import tpu_sc as plsc

N = 10000
E = 320000
D_IN = 128
D_HID = 128
D_OUT = 64
DH = D_HID // 2

NC = 2
NS = 16
NW = NC * NS
K = 128
NCH = 80
E_PAD = NW * NCH * K
TCH = E_PAD // (NS * K)
CPP = 40
NPAD = N + 8
RPT = 624
REM = N - RPT * NS

_f32 = jnp.float32


def _zero_acc(s, zrow, acc):
  off = pl.multiple_of(s * RPT, 8)
  pltpu.sync_copy(zrow.at[pl.ds(off, RPT)], acc.at[pl.ds(off, RPT)])

  @pl.when(s == NS - 1)
  def _():
    pltpu.sync_copy(zrow.at[pl.ds(0, REM + 8)],
                    acc.at[pl.ds(RPT * NS, REM + 8)])

  return off


def _stage_table(s, zh, table):
  off = pl.multiple_of(s * RPT, 8)
  pltpu.sync_copy(zh.at[pl.ds(off, RPT)], table.at[pl.ds(off, RPT)])

  @pl.when(s == NS - 1)
  def _():
    pltpu.sync_copy(zh.at[pl.ds(RPT * NS, REM)],
                    table.at[pl.ds(RPT * NS, REM)])


def _copy_out(c, s, off, acc, out):
  pltpu.sync_copy(acc.at[pl.ds(off, RPT)], out.at[c, pl.ds(off, RPT)])

  @pl.when(s == NS - 1)
  def _():
    pltpu.sync_copy(acc.at[pl.ds(RPT * NS, REM)],
                    out.at[c, pl.ds(RPT * NS, REM)])


def _agg_loop(tables, acc, src_idx, dst_idx, srcv, dstv, rowsv, gsem,
              n_chunks, idx_row):
  for p in range(n_chunks // CPP):
    pltpu.sync_copy(src_idx.at[idx_row, pl.ds(p * CPP, CPP)], srcv)
    pltpu.sync_copy(dst_idx.at[idx_row, pl.ds(p * CPP, CPP)], dstv)
    pltpu.async_copy(tables[0].at[srcv.at[0]], rowsv.at[0], gsem[0])
    pltpu.async_copy(tables[1].at[srcv.at[1]], rowsv.at[1], gsem[1])

    def pair(j, carry):
      for b in range(2):
        i = j * 2 + b
        pltpu.make_async_copy(tables[b].at[srcv.at[i]], rowsv.at[b],
                              gsem[b]).wait()
        pltpu.sync_copy(rowsv.at[b], acc.at[dstv.at[i]], add=True)

        @pl.when(i + 2 < CPP)
        def _():
          pltpu.async_copy(tables[b].at[srcv.at[i + 2]], rowsv.at[b],
                           gsem[b])

      return carry

    lax.fori_loop(0, CPP // 2, pair, 0)


def _make_sc_agg_hbm128():
  mesh = plsc.VectorSubcoreMesh(core_axis_name="c", subcore_axis_name="s")
  out_type = [jax.ShapeDtypeStruct((NC, N, D_HID), _f32)]
  scratch = [
      pltpu.VMEM((CPP, K), jnp.int32),
      pltpu.VMEM((CPP, K), jnp.int32),
      pltpu.VMEM((2, K, D_HID), _f32),
      pltpu.VMEM_SHARED((NPAD, D_HID), _f32),
      pltpu.SemaphoreType.DMA,
      pltpu.SemaphoreType.DMA,
  ]

  def body(za, zb, src3, dst3, zrow, agg_out, srcv, dstv, rowsv, acc,
           g0, g1):
    c = lax.axis_index("c")
    s = lax.axis_index("s")
    wid = c * NS + s
    off = _zero_acc(s, zrow, acc)
    plsc.subcore_barrier()
    _agg_loop((za, zb), acc, src3, dst3, srcv, dstv, rowsv, (g0, g1),
              NCH, wid)
    plsc.subcore_barrier()
    _copy_out(c, s, off, acc, agg_out)

  return pl.kernel(body, mesh=mesh, out_type=out_type, scratch_types=scratch)


def _make_sc_agg_hbm64():
  mesh = plsc.VectorSubcoreMesh(core_axis_name="c", subcore_axis_name="s")
  out_type = [jax.ShapeDtypeStruct((NC, N, DH), _f32)]
  scratch = [
      pltpu.VMEM((CPP, K), jnp.int32),
      pltpu.VMEM((CPP, K), jnp.int32),
      pltpu.VMEM((2, K, DH), _f32),
      pltpu.VMEM_SHARED((NPAD, DH), _f32),
      pltpu.SemaphoreType.DMA,
      pltpu.SemaphoreType.DMA,
  ]

  def body(za, zb, src3, dst3, zrow, agg_out, srcv, dstv, rowsv, acc,
           g0, g1):
    c = lax.axis_index("c")
    s = lax.axis_index("s")
    wid = c * NS + s
    off = _zero_acc(s, zrow, acc)
    plsc.subcore_barrier()
    _agg_loop((za, zb), acc, src3, dst3, srcv, dstv, rowsv, (g0, g1),
              NCH, wid)
    plsc.subcore_barrier()
    _copy_out(c, s, off, acc, agg_out)

  return pl.kernel(
      body, mesh=mesh, out_type=out_type, scratch_types=scratch,
      compiler_params=pltpu.CompilerParams(use_tc_tiling_on_sc=False))


def _make_sc_cnt():
  mesh = plsc.VectorSubcoreMesh(core_axis_name="c", subcore_axis_name="s")
  out_type = [jax.ShapeDtypeStruct((NC, N, D_HID), _f32)]
  scratch = [
      pltpu.VMEM((NCH, K), jnp.int32),
      pltpu.VMEM((K, D_HID), _f32),
      pltpu.VMEM_SHARED((NPAD, D_HID), _f32),
  ]

  def body(dst3, zrow, one_tab, cnt_out, dstv, onesv, acc):
    c = lax.axis_index("c")
    s = lax.axis_index("s")
    wid = c * NS + s
    pltpu.sync_copy(dst3.at[wid], dstv)
    pltpu.sync_copy(one_tab, onesv)
    off = _zero_acc(s, zrow, acc)
    plsc.subcore_barrier()

    def chunk(i, carry):
      pltpu.sync_copy(onesv, acc.at[dstv.at[i]], add=True)
      return carry

    lax.fori_loop(0, NCH, chunk, 0)
    plsc.subcore_barrier()
    _copy_out(c, s, off, acc, cnt_out)

  return pl.kernel(body, mesh=mesh, out_type=out_type, scratch_types=scratch)


_sc_agg_hbm128 = functools.lru_cache(maxsize=None)(_make_sc_agg_hbm128)
_sc_agg_hbm64 = functools.lru_cache(maxsize=None)(_make_sc_agg_hbm64)
_sc_cnt = functools.lru_cache(maxsize=None)(_make_sc_cnt)


BN = 1000


def _row_spec(d):
  return pl.BlockSpec((BN, d), lambda i: (i, 0))


def _full_spec(shape):
  return pl.BlockSpec(shape, lambda i: (0, 0))


def _dot(a, b):
  return jnp.dot(a, b, preferred_element_type=_f32,
                 precision=lax.Precision.HIGHEST)


def _ln(pre, g, be):
  mu = jnp.mean(pre, axis=-1, keepdims=True)
  dv = pre - mu
  var = jnp.mean(dv * dv, axis=-1, keepdims=True)
  return dv / jnp.sqrt(var + 1e-5) * g + be


def _tc_pre(x, wl, wr, bl):
  d_in, d_out = wl.shape

  def body(x_r, wl_r, wr_r, bl_r, za_r, zb_r, r_r):
    xv = x_r[...]
    z = _dot(xv, wl_r[...])
    za_r[...] = z
    zb_r[...] = z
    r_r[...] = _dot(xv, wr_r[...]) + bl_r[...]

  return pl.pallas_call(
      body,
      grid=(N // BN,),
      in_specs=[_row_spec(d_in), _full_spec(wl.shape), _full_spec(wr.shape),
                _full_spec((1, d_out))],
      out_specs=[_row_spec(d_out), _row_spec(d_out), _row_spec(d_out)],
      out_shape=[jax.ShapeDtypeStruct((N, d_out), _f32)] * 3,
  )(x, wl, wr, bl.reshape(1, -1))


def _tc_mid(aggA, aggB, cnta, cntb, r, g, be, wl, wr, bl):
  d = r.shape[1]
  da = aggA.shape[1]
  dz = wl.shape[1]
  dr = wr.shape[1]

  def body(aa_r, ab_r, ca_r, cb_r, r_r, g_r, be_r, wl_r, wr_r, bl_r,
           za_r, zb_r, rn_r):
    cnt = jnp.maximum(ca_r[...] + cb_r[...], 1.0)
    pre = (aa_r[...] + ab_r[...]) / cnt + r_r[...]
    h = jnp.maximum(_ln(pre, g_r[...], be_r[...]), 0.0)
    z = _dot(h, wl_r[...])
    za_r[...] = z
    zb_r[...] = z
    rn_r[...] = _dot(h, wr_r[...]) + bl_r[...]

  return pl.pallas_call(
      body,
      grid=(N // BN,),
      in_specs=[_row_spec(da), _row_spec(da),
                pl.BlockSpec((BN, 1), lambda i: (i, 0)),
                pl.BlockSpec((BN, 1), lambda i: (i, 0)),
                _row_spec(d), _full_spec((1, d)), _full_spec((1, d)),
                _full_spec(wl.shape), _full_spec(wr.shape),
                _full_spec((1, dr))],
      out_specs=[_row_spec(dz), _row_spec(dz), _row_spec(dr)],
      out_shape=[jax.ShapeDtypeStruct((N, dz), _f32),
                 jax.ShapeDtypeStruct((N, dz), _f32),
                 jax.ShapeDtypeStruct((N, dr), _f32)],
  )(aggA, aggB, cnta, cntb, r, g.reshape(1, -1), be.reshape(1, -1),
    wl, wr, bl.reshape(1, -1))


def _tc_post(aggA, aggB, cnta, cntb, r, g, be):
  do = r.shape[1]

  def body(aa_r, ab_r, ca_r, cb_r, r_r, g_r, be_r, o_r):
    cnt = jnp.maximum(ca_r[...] + cb_r[...], 1.0)
    pre = (aa_r[...] + ab_r[...]) / cnt + r_r[...]
    h = _ln(pre, g_r[...], be_r[...])
    nrm = jnp.sqrt(jnp.sum(h * h, axis=-1, keepdims=True))
    o_r[...] = h / jnp.maximum(nrm, 1e-12)

  return pl.pallas_call(
      body,
      grid=(N // BN,),
      in_specs=[_row_spec(do), _row_spec(do),
                pl.BlockSpec((BN, 1), lambda i: (i, 0)),
                pl.BlockSpec((BN, 1), lambda i: (i, 0)),
                _row_spec(do), _full_spec((1, do)), _full_spec((1, do))],
      out_specs=_row_spec(do),
      out_shape=jax.ShapeDtypeStruct((N, do), _f32),
  )(aggA, aggB, cnta, cntb, r, g.reshape(1, -1), be.reshape(1, -1))


def kernel(x, edge_index, Wl0, bl0, Wr0, g0, be0, Wl1, bl1, Wr1, g1, be1,
           Wl2, bl2, Wr2, g2, be2):
  npad = E_PAD - E
  srcp = jnp.concatenate([edge_index[0], jnp.zeros((npad,), jnp.int32)])
  dstp = jnp.concatenate([edge_index[1], jnp.full((npad,), N, jnp.int32)])
  src3 = srcp.reshape(NW, NCH, K)
  dst3 = dstp.reshape(NW, NCH, K)
  ones_blk = jnp.ones((K, D_HID), _f32)

  (cntw,) = _sc_cnt()(dst3, jnp.zeros((N, D_HID), _f32), ones_blk)
  cnta = cntw[0, :, 0:1]
  cntb = cntw[1, :, 0:1]
  zrow128 = cntw[0] * 0.0
  zrow64 = zrow128[:, :DH]

  z0a, z0b, r0 = _tc_pre(x, Wl0, Wr0, bl0)
  (agg0,) = _sc_agg_hbm128()(z0a, z0b, src3, dst3, zrow128)
  z1a, z1b, r1 = _tc_mid(agg0[0], agg0[1], cnta, cntb, r0, g0, be0,
                         Wl1, Wr1, bl1)

  (agg1,) = _sc_agg_hbm128()(z1a, z1b, src3, dst3, zrow128)
  z2a, z2b, r2 = _tc_mid(agg1[0], agg1[1], cnta, cntb, r1, g1, be1,
                         Wl2, Wr2, bl2)

  (agg2,) = _sc_agg_hbm64()(z2a, z2b, src3, dst3, zrow64)
  return _tc_post(agg2[0], agg2[1], cnta, cntb, r2, g2, be2)


# speedup vs baseline: 1.7588x; 1.0169x over previous
import functools

import jax
import jax.numpy as jnp
from jax import lax
from jax.experimental import pallas as pl
from jax.experimental.pallas import tpu as pltpu
from jax.experimental.pallas import tpu_sc as plsc

N = 10000
E = 320000
D_IN = 128
D_HID = 128
D_OUT = 64
DH = D_HID // 2

NC = 2
NS = 16
NW = NC * NS
K = 128
NCH = 80
E_PAD = NW * NCH * K
TCH = E_PAD // (NS * K)
CPP = 40
NPAD = N + 8
RPT = 624
REM = N - RPT * NS

_f32 = jnp.float32


def _zero_acc(s, zrow, acc):
  off = pl.multiple_of(s * RPT, 8)
  pltpu.sync_copy(zrow.at[pl.ds(off, RPT)], acc.at[pl.ds(off, RPT)])

  @pl.when(s == NS - 1)
  def _():
    pltpu.sync_copy(zrow.at[pl.ds(0, REM + 8)],
                    acc.at[pl.ds(RPT * NS, REM + 8)])

  return off


def _stage_table(s, zh, table):
  off = pl.multiple_of(s * RPT, 8)
  pltpu.sync_copy(zh.at[pl.ds(off, RPT)], table.at[pl.ds(off, RPT)])

  @pl.when(s == NS - 1)
  def _():
    pltpu.sync_copy(zh.at[pl.ds(RPT * NS, REM)],
                    table.at[pl.ds(RPT * NS, REM)])


def _copy_out(c, s, off, acc, out):
  pltpu.sync_copy(acc.at[pl.ds(off, RPT)], out.at[c, pl.ds(off, RPT)])

  @pl.when(s == NS - 1)
  def _():
    pltpu.sync_copy(acc.at[pl.ds(RPT * NS, REM)],
                    out.at[c, pl.ds(RPT * NS, REM)])


def _agg_loop(tables, acc, src_idx, dst_idx, srcv, dstv, rowsv, gsem,
              n_chunks, idx_row):
  for p in range(n_chunks // CPP):
    pltpu.sync_copy(src_idx.at[idx_row, pl.ds(p * CPP, CPP)], srcv)
    pltpu.sync_copy(dst_idx.at[idx_row, pl.ds(p * CPP, CPP)], dstv)
    pltpu.async_copy(tables[0].at[srcv.at[0]], rowsv.at[0], gsem[0])
    pltpu.async_copy(tables[1].at[srcv.at[1]], rowsv.at[1], gsem[1])

    def quad(j, carry):
      for b4 in range(4):
        i = j * 4 + b4
        b = b4 % 2
        pltpu.make_async_copy(tables[b4].at[srcv.at[i]], rowsv.at[b],
                              gsem[b]).wait()
        pltpu.sync_copy(rowsv.at[b], acc.at[dstv.at[i]], add=True)

        @pl.when(i + 2 < CPP)
        def _():
          pltpu.async_copy(tables[(b4 + 2) % 4].at[srcv.at[i + 2]],
                           rowsv.at[b], gsem[b])

      return carry

    lax.fori_loop(0, CPP // 4, quad, 0)


def _make_sc_agg_hbm128():
  mesh = plsc.VectorSubcoreMesh(core_axis_name="c", subcore_axis_name="s")
  out_type = [jax.ShapeDtypeStruct((NC, N, D_HID), _f32)]
  scratch = [
      pltpu.VMEM((CPP, K), jnp.int32),
      pltpu.VMEM((CPP, K), jnp.int32),
      pltpu.VMEM((2, K, D_HID), _f32),
      pltpu.VMEM_SHARED((NPAD, D_HID), _f32),
      pltpu.SemaphoreType.DMA,
      pltpu.SemaphoreType.DMA,
  ]

  def body(za, zb, zc, zd, src3, dst3, zrow, agg_out, srcv, dstv, rowsv,
           acc, g0, g1):
    c = lax.axis_index("c")
    s = lax.axis_index("s")
    wid = c * NS + s
    off = _zero_acc(s, zrow, acc)
    plsc.subcore_barrier()
    _agg_loop((za, zb, zc, zd), acc, src3, dst3, srcv, dstv, rowsv,
              (g0, g1), NCH, wid)
    plsc.subcore_barrier()
    _copy_out(c, s, off, acc, agg_out)

  return pl.kernel(body, mesh=mesh, out_type=out_type, scratch_types=scratch)


def _make_sc_agg_hbm64():
  mesh = plsc.VectorSubcoreMesh(core_axis_name="c", subcore_axis_name="s")
  out_type = [jax.ShapeDtypeStruct((NC, N, DH), _f32)]
  scratch = [
      pltpu.VMEM((CPP, K), jnp.int32),
      pltpu.VMEM((CPP, K), jnp.int32),
      pltpu.VMEM((2, K, DH), _f32),
      pltpu.VMEM_SHARED((NPAD, DH), _f32),
      pltpu.SemaphoreType.DMA,
      pltpu.SemaphoreType.DMA,
  ]

  def body(za, zb, zc, zd, src3, dst3, zrow, agg_out, srcv, dstv, rowsv,
           acc, g0, g1):
    c = lax.axis_index("c")
    s = lax.axis_index("s")
    wid = c * NS + s
    off = _zero_acc(s, zrow, acc)
    plsc.subcore_barrier()
    _agg_loop((za, zb, zc, zd), acc, src3, dst3, srcv, dstv, rowsv,
              (g0, g1), NCH, wid)
    plsc.subcore_barrier()
    _copy_out(c, s, off, acc, agg_out)

  return pl.kernel(
      body, mesh=mesh, out_type=out_type, scratch_types=scratch,
      compiler_params=pltpu.CompilerParams(use_tc_tiling_on_sc=False))


def _make_sc_cnt():
  mesh = plsc.VectorSubcoreMesh(core_axis_name="c", subcore_axis_name="s")
  out_type = [jax.ShapeDtypeStruct((NC, N, D_HID), _f32)]
  scratch = [
      pltpu.VMEM((NCH, K), jnp.int32),
      pltpu.VMEM((K, D_HID), _f32),
      pltpu.VMEM_SHARED((NPAD, D_HID), _f32),
  ]

  def body(dst3, zrow, one_tab, cnt_out, dstv, onesv, acc):
    c = lax.axis_index("c")
    s = lax.axis_index("s")
    wid = c * NS + s
    pltpu.sync_copy(dst3.at[wid], dstv)
    pltpu.sync_copy(one_tab, onesv)
    off = _zero_acc(s, zrow, acc)
    plsc.subcore_barrier()

    def chunk(i, carry):
      pltpu.sync_copy(onesv, acc.at[dstv.at[i]], add=True)
      return carry

    lax.fori_loop(0, NCH, chunk, 0)
    plsc.subcore_barrier()
    _copy_out(c, s, off, acc, cnt_out)

  return pl.kernel(body, mesh=mesh, out_type=out_type, scratch_types=scratch)


_sc_agg_hbm128 = functools.lru_cache(maxsize=None)(_make_sc_agg_hbm128)
_sc_agg_hbm64 = functools.lru_cache(maxsize=None)(_make_sc_agg_hbm64)
_sc_cnt = functools.lru_cache(maxsize=None)(_make_sc_cnt)


BN = 1000


def _row_spec(d):
  return pl.BlockSpec((BN, d), lambda i: (i, 0))


def _full_spec(shape):
  return pl.BlockSpec(shape, lambda i: (0, 0))


def _dot(a, b):
  return jnp.dot(a, b, preferred_element_type=_f32,
                 precision=lax.Precision.HIGHEST)


def _ln(pre, g, be):
  mu = jnp.mean(pre, axis=-1, keepdims=True)
  dv = pre - mu
  var = jnp.mean(dv * dv, axis=-1, keepdims=True)
  return dv / jnp.sqrt(var + 1e-5) * g + be


def _tc_pre(x, wl, wr, bl):
  d_in, d_out = wl.shape

  def body(x_r, wl_r, wr_r, bl_r, *outs):
    xv = x_r[...]
    z = _dot(xv, wl_r[...])
    for zo in outs[:4]:
      zo[...] = z
    outs[4][...] = _dot(xv, wr_r[...]) + bl_r[...]

  return pl.pallas_call(
      body,
      grid=(N // BN,),
      in_specs=[_row_spec(d_in), _full_spec(wl.shape), _full_spec(wr.shape),
                _full_spec((1, d_out))],
      out_specs=[_row_spec(d_out)] * 5,
      out_shape=[jax.ShapeDtypeStruct((N, d_out), _f32)] * 5,
  )(x, wl, wr, bl.reshape(1, -1))


def _tc_mid(aggA, aggB, cnta, cntb, r, g, be, wl, wr, bl):
  d = r.shape[1]
  da = aggA.shape[1]
  dz = wl.shape[1]
  dr = wr.shape[1]

  def body(aa_r, ab_r, ca_r, cb_r, r_r, g_r, be_r, wl_r, wr_r, bl_r,
           *outs):
    cnt = jnp.maximum(ca_r[...] + cb_r[...], 1.0)
    pre = (aa_r[...] + ab_r[...]) / cnt + r_r[...]
    h = jnp.maximum(_ln(pre, g_r[...], be_r[...]), 0.0)
    z = _dot(h, wl_r[...])
    for zo in outs[:4]:
      zo[...] = z
    outs[4][...] = _dot(h, wr_r[...]) + bl_r[...]

  return pl.pallas_call(
      body,
      grid=(N // BN,),
      in_specs=[_row_spec(da), _row_spec(da),
                pl.BlockSpec((BN, 1), lambda i: (i, 0)),
                pl.BlockSpec((BN, 1), lambda i: (i, 0)),
                _row_spec(d), _full_spec((1, d)), _full_spec((1, d)),
                _full_spec(wl.shape), _full_spec(wr.shape),
                _full_spec((1, dr))],
      out_specs=[_row_spec(dz)] * 4 + [_row_spec(dr)],
      out_shape=[jax.ShapeDtypeStruct((N, dz), _f32)] * 4
                + [jax.ShapeDtypeStruct((N, dr), _f32)],
  )(aggA, aggB, cnta, cntb, r, g.reshape(1, -1), be.reshape(1, -1),
    wl, wr, bl.reshape(1, -1))


def _tc_post(aggA, aggB, cnta, cntb, r, g, be):
  do = r.shape[1]

  def body(aa_r, ab_r, ca_r, cb_r, r_r, g_r, be_r, o_r):
    cnt = jnp.maximum(ca_r[...] + cb_r[...], 1.0)
    pre = (aa_r[...] + ab_r[...]) / cnt + r_r[...]
    h = _ln(pre, g_r[...], be_r[...])
    nrm = jnp.sqrt(jnp.sum(h * h, axis=-1, keepdims=True))
    o_r[...] = h / jnp.maximum(nrm, 1e-12)

  return pl.pallas_call(
      body,
      grid=(N // BN,),
      in_specs=[_row_spec(do), _row_spec(do),
                pl.BlockSpec((BN, 1), lambda i: (i, 0)),
                pl.BlockSpec((BN, 1), lambda i: (i, 0)),
                _row_spec(do), _full_spec((1, do)), _full_spec((1, do))],
      out_specs=_row_spec(do),
      out_shape=jax.ShapeDtypeStruct((N, do), _f32),
  )(aggA, aggB, cnta, cntb, r, g.reshape(1, -1), be.reshape(1, -1))


def kernel(x, edge_index, Wl0, bl0, Wr0, g0, be0, Wl1, bl1, Wr1, g1, be1,
           Wl2, bl2, Wr2, g2, be2):
  npad = E_PAD - E
  srcp = jnp.concatenate([edge_index[0], jnp.zeros((npad,), jnp.int32)])
  dstp = jnp.concatenate([edge_index[1], jnp.full((npad,), N, jnp.int32)])
  src3 = srcp.reshape(NW, NCH, K)
  dst3 = dstp.reshape(NW, NCH, K)
  ones_blk = jnp.ones((K, D_HID), _f32)

  (cntw,) = _sc_cnt()(dst3, jnp.zeros((N, D_HID), _f32), ones_blk)
  cnta = cntw[0, :, 0:1]
  cntb = cntw[1, :, 0:1]
  zrow128 = cntw[0] * 0.0
  zrow64 = zrow128[:, :DH]

  z0a, z0b, z0c, z0d, r0 = _tc_pre(x, Wl0, Wr0, bl0)
  (agg0,) = _sc_agg_hbm128()(z0a, z0b, z0c, z0d, src3, dst3, zrow128)
  z1a, z1b, z1c, z1d, r1 = _tc_mid(agg0[0], agg0[1], cnta, cntb, r0,
                                   g0, be0, Wl1, Wr1, bl1)

  (agg1,) = _sc_agg_hbm128()(z1a, z1b, z1c, z1d, src3, dst3, zrow128)
  z2a, z2b, z2c, z2d, r2 = _tc_mid(agg1[0], agg1[1], cnta, cntb, r1,
                                   g1, be1, Wl2, Wr2, bl2)

  (agg2,) = _sc_agg_hbm64()(z2a, z2b, z2c, z2d, src3, dst3, zrow64)
  return _tc_post(agg2[0], agg2[1], cnta, cntb, r2, g2, be2)
